# baseline (speedup 1.0000x reference)
import functools

import jax
import jax.numpy as jnp
from jax import lax
from jax.experimental import pallas as pl
from jax.experimental.pallas import tpu as pltpu
from jax.experimental.pallas import tpu_sc as plsc

N_NODES = 10000
N_EDGES = 160000
CHUNK = 128
N_WORKERS = 32
CH_PER_W = 40
E_PAD = N_WORKERS * CH_PER_W * CHUNK
N_CHUNKS = E_PAD // CHUNK
CH_PER_T = N_CHUNKS // 16
ACC_ROWS = 10112
ROWS_PER_TILE = ACC_ROWS // 16
W_ACC = 16

N_ADJ_BM = 256
N_SCORE_BM = 4096

_SC_PARAMS = pltpu.CompilerParams(use_tc_tiling_on_sc=False)


@functools.lru_cache(maxsize=None)
def _gather_sc():
    mesh = plsc.VectorSubcoreMesh(core_axis_name="c", subcore_axis_name="s")

    @functools.partial(
        pl.kernel,
        out_type=[jax.ShapeDtypeStruct((E_PAD, 16), jnp.float32),
                  jax.ShapeDtypeStruct((E_PAD, 16), jnp.float32)],
        mesh=mesh,
        compiler_params=_SC_PARAMS,
        scratch_types=[
            pltpu.VMEM((CH_PER_W, CHUNK), jnp.int32),
            pltpu.VMEM((CH_PER_W, CHUNK), jnp.int32),
            pltpu.VMEM((2, CHUNK, 16), jnp.float32),
            pltpu.VMEM((2, CHUNK, 16), jnp.float32),
            pltpu.SemaphoreType.DMA((2,)),
            pltpu.SemaphoreType.DMA((2,)),
        ],
    )
    def k(tab, src2d, dst2d, ga, gb, src_v, dst_v, a_v, b_v, sem_g, sem_s):
        c = lax.axis_index("c")
        s = lax.axis_index("s")
        wid = c * 16 + s
        ch0 = wid * CH_PER_W
        pltpu.sync_copy(src2d.at[pl.ds(ch0, CH_PER_W)], src_v)
        pltpu.sync_copy(dst2d.at[pl.ds(ch0, CH_PER_W)], dst_v)
        base0 = ch0 * CHUNK

        def fire_gather(j):
            b = j & 1
            pltpu.async_copy(tab.at[src_v.at[j]], a_v.at[b], sem_g.at[b])
            pltpu.async_copy(tab.at[dst_v.at[j]], b_v.at[b], sem_g.at[b])

        def wait_gather(j):
            b = j & 1
            pltpu.make_async_copy(tab.at[src_v.at[j]], a_v.at[b],
                                  sem_g.at[b]).wait()
            pltpu.make_async_copy(tab.at[dst_v.at[j]], b_v.at[b],
                                  sem_g.at[b]).wait()

        def fire_store(j):
            b = j & 1
            sl = pl.ds(base0 + j * CHUNK, CHUNK)
            pltpu.async_copy(a_v.at[b], ga.at[sl], sem_s.at[b])
            pltpu.async_copy(b_v.at[b], gb.at[sl], sem_s.at[b])

        def wait_store(j):
            b = j & 1
            sl = pl.ds(base0 + j * CHUNK, CHUNK)
            pltpu.make_async_copy(a_v.at[b], ga.at[sl], sem_s.at[b]).wait()
            pltpu.make_async_copy(b_v.at[b], gb.at[sl], sem_s.at[b]).wait()

        fire_gather(0)

        @pl.loop(0, CH_PER_W)
        def _chunk(j):
            @pl.when(j + 1 < CH_PER_W)
            def _():
                @pl.when(j >= 1)
                def _():
                    wait_store(j - 1)
                fire_gather(j + 1)

            wait_gather(j)
            fire_store(j)

        wait_store(CH_PER_W - 2)
        wait_store(CH_PER_W - 1)

    return k


def _scatter_phase(vals, dst_v, accum, v_v, sem_ld, sem_sc, s):
    base0 = s * CH_PER_T * CHUNK

    def fire_load(j):
        b = j & 1
        pltpu.async_copy(vals.at[pl.ds(base0 + j * CHUNK, CHUNK)],
                         v_v.at[b], sem_ld.at[b])

    def wait_load(j):
        b = j & 1
        pltpu.make_async_copy(vals.at[pl.ds(base0 + j * CHUNK, CHUNK)],
                              v_v.at[b], sem_ld.at[b]).wait()

    def fire_scatter(j):
        b = j & 1
        pltpu.async_copy(v_v.at[b], accum.at[dst_v.at[j]],
                         sem_sc.at[b], add=True)

    def wait_scatter(j):
        b = j & 1
        pltpu.make_async_copy(v_v.at[b], accum.at[dst_v.at[j]],
                              sem_sc.at[b]).wait()

    fire_load(0)

    @pl.loop(0, CH_PER_T)
    def _chunk(j):
        @pl.when(j + 1 < CH_PER_T)
        def _():
            @pl.when(j >= 1)
            def _():
                wait_scatter(j - 1)
            fire_load(j + 1)

        wait_load(j)
        fire_scatter(j)

    wait_scatter(CH_PER_T - 2)
    wait_scatter(CH_PER_T - 1)


@functools.lru_cache(maxsize=None)
def _merged_sc():
    mesh = plsc.VectorSubcoreMesh(core_axis_name="c", subcore_axis_name="s")

    @functools.partial(
        pl.kernel,
        out_type=[jax.ShapeDtypeStruct((2 * ACC_ROWS, 16), jnp.float32),
                  jax.ShapeDtypeStruct((E_PAD, 16), jnp.float32),
                  jax.ShapeDtypeStruct((E_PAD, 16), jnp.float32)],
        mesh=mesh,
        compiler_params=_SC_PARAMS,
        scratch_types=[
            pltpu.VMEM((CH_PER_T, CHUNK), jnp.int32),
            pltpu.VMEM((CH_PER_W, CHUNK), jnp.int32),
            pltpu.VMEM((CH_PER_W, CHUNK), jnp.int32),
            pltpu.VMEM((2, CHUNK, 16), jnp.float32),
            pltpu.VMEM((2, CHUNK, 16), jnp.float32),
            pltpu.VMEM((ROWS_PER_TILE, 16), jnp.float32),
            pltpu.VMEM_SHARED((ACC_ROWS, W_ACC), jnp.float32),
            pltpu.SemaphoreType.DMA((2,)),
            pltpu.SemaphoreType.DMA((2,)),
        ],
    )
    def k(vals, dsts2d, srcg2d, dstg2d, tab, ga, gb,
          dsts_v, srcg_v, dstg_v, a_v, b_v, bounce_v, accum, sem1, sem2):
        c = lax.axis_index("c")
        s = lax.axis_index("s")
        wid = c * 16 + s
        zeros16 = jnp.zeros((16,), jnp.float32)

        @pl.loop(0, ROWS_PER_TILE)
        def _zb(i):
            bounce_v[i, :] = zeros16

        row0 = s * ROWS_PER_TILE
        pltpu.sync_copy(bounce_v, accum.at[pl.ds(row0, ROWS_PER_TILE)])
        pltpu.sync_copy(dsts2d.at[pl.ds(s * CH_PER_T, CH_PER_T)], dsts_v)
        ch0 = wid * CH_PER_W
        pltpu.sync_copy(srcg2d.at[pl.ds(ch0, CH_PER_W)], srcg_v)
        pltpu.sync_copy(dstg2d.at[pl.ds(ch0, CH_PER_W)], dstg_v)
        plsc.subcore_barrier()

        _scatter_phase(vals, dsts_v, accum, a_v, sem1, sem2, s)

        plsc.subcore_barrier()
        pltpu.sync_copy(accum.at[pl.ds(row0, ROWS_PER_TILE)], bounce_v)
        pltpu.sync_copy(bounce_v,
                        tab.at[pl.ds(c * ACC_ROWS + row0, ROWS_PER_TILE)])
        plsc.subcore_barrier()

        base0 = ch0 * CHUNK

        def fire_gather(j):
            b = j & 1
            pltpu.async_copy(tab.at[srcg_v.at[j]], a_v.at[b], sem1.at[b])
            pltpu.async_copy(tab.at[dstg_v.at[j]], b_v.at[b], sem1.at[b])

        def wait_gather(j):
            b = j & 1
            pltpu.make_async_copy(tab.at[srcg_v.at[j]], a_v.at[b],
                                  sem1.at[b]).wait()
            pltpu.make_async_copy(tab.at[dstg_v.at[j]], b_v.at[b],
                                  sem1.at[b]).wait()

        def fire_store(j):
            b = j & 1
            sl = pl.ds(base0 + j * CHUNK, CHUNK)
            pltpu.async_copy(a_v.at[b], ga.at[sl], sem2.at[b])
            pltpu.async_copy(b_v.at[b], gb.at[sl], sem2.at[b])

        def wait_store(j):
            b = j & 1
            sl = pl.ds(base0 + j * CHUNK, CHUNK)
            pltpu.make_async_copy(a_v.at[b], ga.at[sl], sem2.at[b]).wait()
            pltpu.make_async_copy(b_v.at[b], gb.at[sl], sem2.at[b]).wait()

        fire_gather(0)

        @pl.loop(0, CH_PER_W)
        def _chunk(j):
            @pl.when(j + 1 < CH_PER_W)
            def _():
                @pl.when(j >= 1)
                def _():
                    wait_store(j - 1)
                fire_gather(j + 1)

            wait_gather(j)
            fire_store(j)

        wait_store(CH_PER_W - 2)
        wait_store(CH_PER_W - 1)

    return k


@functools.lru_cache(maxsize=None)
def _scatter_sc():
    mesh = plsc.VectorSubcoreMesh(core_axis_name="c", subcore_axis_name="s")

    @functools.partial(
        pl.kernel,
        out_type=jax.ShapeDtypeStruct((2, ACC_ROWS, W_ACC), jnp.float32),
        mesh=mesh,
        compiler_params=_SC_PARAMS,
        scratch_types=[
            pltpu.VMEM((CH_PER_W, CHUNK), jnp.int32),
            pltpu.VMEM((2, CHUNK, 16), jnp.float32),
            pltpu.VMEM((ROWS_PER_TILE, 16), jnp.float32),
            pltpu.VMEM_SHARED((ACC_ROWS, W_ACC), jnp.float32),
            pltpu.SemaphoreType.DMA((2,)),
            pltpu.SemaphoreType.DMA((2,)),
        ],
    )
    def k(vals, dst2d, out, dst_v, v_v, bounce_v, accum, sem_ld, sem_sc):
        c = lax.axis_index("c")
        s = lax.axis_index("s")
        wid = c * 16 + s
        zeros16 = jnp.zeros((16,), jnp.float32)

        @pl.loop(0, ROWS_PER_TILE)
        def _zb(i):
            bounce_v[i, :] = zeros16

        row0 = s * ROWS_PER_TILE
        pltpu.sync_copy(bounce_v, accum.at[pl.ds(row0, ROWS_PER_TILE)])
        plsc.subcore_barrier()

        ch0 = wid * CH_PER_W
        pltpu.sync_copy(dst2d.at[pl.ds(ch0, CH_PER_W)], dst_v)
        base0 = ch0 * CHUNK

        def fire_load(j):
            b = j & 1
            pltpu.async_copy(vals.at[pl.ds(base0 + j * CHUNK, CHUNK)],
                             v_v.at[b], sem_ld.at[b])

        def wait_load(j):
            b = j & 1
            pltpu.make_async_copy(vals.at[pl.ds(base0 + j * CHUNK, CHUNK)],
                                  v_v.at[b], sem_ld.at[b]).wait()

        def fire_scatter(j):
            b = j & 1
            pltpu.async_copy(v_v.at[b], accum.at[dst_v.at[j]],
                             sem_sc.at[b], add=True)

        def wait_scatter(j):
            b = j & 1
            pltpu.make_async_copy(v_v.at[b], accum.at[dst_v.at[j]],
                                  sem_sc.at[b]).wait()

        fire_load(0)

        @pl.loop(0, CH_PER_W)
        def _chunk(j):
            @pl.when(j + 1 < CH_PER_W)
            def _():
                @pl.when(j >= 1)
                def _():
                    wait_scatter(j - 1)
                fire_load(j + 1)

            wait_load(j)
            fire_scatter(j)

        wait_scatter(CH_PER_W - 2)
        wait_scatter(CH_PER_W - 1)
        plsc.subcore_barrier()
        pltpu.sync_copy(accum.at[pl.ds(row0, ROWS_PER_TILE)], bounce_v)
        pltpu.sync_copy(bounce_v, out.at[c, pl.ds(row0, ROWS_PER_TILE)])

    return k


def _edge_h(g, f, act):
    if f is None:
        return g
    num = g[:, 0:f]
    den = g[:, f:f + 1]
    h = num / (den + jnp.float32(1e-16))
    if act:
        h = jnp.maximum(h, jnp.float32(0.0))
    cols = [h]
    if f < 15:
        cols.append(jnp.zeros((g.shape[0], 15 - f), jnp.float32))
    cols.append(jnp.ones((g.shape[0], 1), jnp.float32))
    return jnp.concatenate(cols, axis=1)


def _he_body(ea_ref, w_ref, o1_ref, o2_ref, o3_ref, o4_ref):
    h = jnp.dot(ea_ref[...], w_ref[...], preferred_element_type=jnp.float32)
    z1 = jnp.zeros((h.shape[0], 8), jnp.float32)
    o1_ref[...] = jnp.concatenate([h[:, 0:8], z1], axis=1)
    o2_ref[...] = jnp.concatenate([h[:, 8:10], z1, z1[:, 0:6]], axis=1)
    o3_ref[...] = jnp.concatenate([h[:, 10:18], z1], axis=1)
    o4_ref[...] = jnp.concatenate([h[:, 18:33], z1[:, 0:1]], axis=1)


def _he_proj(ea_pad, wecat):
    bm = 4096
    grid = (E_PAD // bm,)
    return pl.pallas_call(
        _he_body,
        grid=grid,
        in_specs=[pl.BlockSpec((bm, 35), lambda i: (i, 0)),
                  pl.BlockSpec((35, 33), lambda i: (0, 0))],
        out_specs=[pl.BlockSpec((bm, 16), lambda i: (i, 0))] * 4,
        out_shape=[jax.ShapeDtypeStruct((E_PAD, 16), jnp.float32)
                   for _ in range(4)],
    )(ea_pad, wecat)


def _make_score_body(f, act, with_mlp):
    def body(ga_ref, gb_ref, he_ref, a_w, b_w, att_ref, *rest):
        if with_mlp:
            p_w, q_w, b1_ref, w2_ref, b2_ref, vals_ref, er_ref = rest
        else:
            vals_ref, = rest
        ha = _edge_h(ga_ref[...], f, act)
        hb = _edge_h(gb_ref[...], f, act)
        he = he_ref[...]
        a2 = jnp.dot(ha, a_w[...], preferred_element_type=jnp.float32)
        b2 = jnp.dot(hb, b_w[...], preferred_element_type=jnp.float32)
        sg = a2 + b2 + he
        u = att_ref[...] * jnp.maximum(sg, jnp.float32(0.2) * sg)
        ex = jnp.exp(jnp.sum(u, axis=1, keepdims=True))
        vals_ref[...] = ex * a2
        if with_mlp:
            zp = jnp.dot(ha, p_w[...], preferred_element_type=jnp.float32)
            zq = jnp.dot(hb, q_w[...], preferred_element_type=jnp.float32)
            hmid = jnp.maximum(zp + zq + b1_ref[...], jnp.float32(0.0))
            er_ref[...] = (jnp.dot(hmid[:, 0:15], w2_ref[...],
                                   preferred_element_type=jnp.float32)
                           + b2_ref[...])
    return body


def _score_tc(ga, gb, he, a_w, b_w, att, f, act, mlp_w=None):
    bm = N_SCORE_BM
    grid = (E_PAD // bm,)
    row_spec = pl.BlockSpec((bm, 16), lambda i: (i, 0))
    w_spec = pl.BlockSpec((16, 16), lambda i: (0, 0))
    v_spec = pl.BlockSpec((1, 16), lambda i: (0, 0))
    in_specs = [row_spec, row_spec, row_spec, w_spec, w_spec, v_spec]
    vals_shape = jax.ShapeDtypeStruct((E_PAD, 16), jnp.float32)
    if mlp_w is None:
        return pl.pallas_call(
            _make_score_body(f, act, False),
            grid=grid,
            in_specs=in_specs,
            out_specs=row_spec,
            out_shape=vals_shape,
        )(ga, gb, he, a_w, b_w, att)
    p_w, q_w, b1, w2, b2 = mlp_w
    return pl.pallas_call(
        _make_score_body(f, act, True),
        grid=grid,
        in_specs=in_specs + [w_spec, w_spec, v_spec,
                             pl.BlockSpec((15, 35), lambda i: (0, 0)),
                             pl.BlockSpec((1, 35), lambda i: (0, 0))],
        out_specs=[row_spec, pl.BlockSpec((bm, 35), lambda i: (i, 0))],
        out_shape=[vals_shape,
                   jax.ShapeDtypeStruct((N_EDGES, 35), jnp.float32)],
    )(ga, gb, he, a_w, b_w, att, p_w, q_w, b1, w2, b2)


def _combine_final(acc, f):
    n = N_NODES

    def body(acc_ref, h_ref):
        num = acc_ref[0, :n, 0:f] + acc_ref[1, :n, 0:f]
        den = acc_ref[0, :n, f:f + 1] + acc_ref[1, :n, f:f + 1]
        h_ref[...] = num / (den + jnp.float32(1e-16))

    return pl.pallas_call(
        body,
        in_specs=[pl.BlockSpec((2, ACC_ROWS, W_ACC), lambda: (0, 0, 0))],
        out_specs=pl.BlockSpec((n, f), lambda: (0, 0)),
        out_shape=jax.ShapeDtypeStruct((n, f), jnp.float32),
    )(acc)


def _acc_z(acc_ref):
    num = acc_ref[0:N_NODES, 0:2]
    den = acc_ref[0:N_NODES, 2:3]
    return num / (den + jnp.float32(1e-16))


def _acc_zb(blk_ref):
    num = blk_ref[:, 0:2]
    den = blk_ref[:, 2:3]
    return num / (den + jnp.float32(1e-16))


def _adj_mean_body(blk_ref, acc_ref, mean_ref):
    z = _acc_z(acc_ref)
    g = jnp.dot(_acc_zb(blk_ref[...]), z.T,
                preferred_element_type=jnp.float32)
    mean_ref[...] = (jnp.sum(g, axis=1) / jnp.float32(N_NODES))[None, :]


def _adj_body(blk_ref, acc_ref, mean_ref, out_ref):
    z = _acc_z(acc_ref)
    g = jnp.dot(_acc_zb(blk_ref[...]), z.T,
                preferred_element_type=jnp.float32)
    out_ref[...] = jnp.tanh(g - mean_ref[...])


def _adj_head(acc2):
    n = N_NODES
    grid = (pl.cdiv(n, N_ADJ_BM),)
    blk_spec = pl.BlockSpec((N_ADJ_BM, 16), lambda i: (i, 0))
    acc_spec = pl.BlockSpec((ACC_ROWS, 16), lambda i: (0, 0))
    means = pl.pallas_call(
        _adj_mean_body,
        grid=grid,
        in_specs=[blk_spec, acc_spec],
        out_specs=pl.BlockSpec((1, N_ADJ_BM), lambda i: (0, i)),
        out_shape=jax.ShapeDtypeStruct((1, n), jnp.float32),
    )(acc2, acc2)
    return pl.pallas_call(
        _adj_body,
        grid=grid,
        in_specs=[blk_spec, acc_spec, pl.BlockSpec((1, n), lambda i: (0, 0))],
        out_specs=pl.BlockSpec((N_ADJ_BM, n), lambda i: (i, 0)),
        out_shape=jax.ShapeDtypeStruct((n, n), jnp.float32),
    )(acc2, acc2, means)


def _wpad(w, marker_col=None):
    m = jnp.zeros((16, 16), jnp.float32)
    m = m.at[:w.shape[0], :w.shape[1]].set(w)
    if marker_col is not None:
        m = m.at[15, marker_col].set(1.0)
    return m


def _vpad(v):
    return jnp.pad(v, (0, 16 - v.shape[0]))[None, :]


def kernel(x, edge_index, edge_attr, params):
    src = edge_index[0]
    dst = edge_index[1]
    npad = E_PAD - N_EDGES
    src_pad = jnp.concatenate(
        [src, jnp.zeros((npad,), jnp.int32)]).reshape(-1, CHUNK)
    dst_pad = jnp.concatenate(
        [dst, jnp.full((npad,), N_NODES, jnp.int32)]).reshape(-1, CHUNK)
    src_g = jnp.concatenate([src_pad[:N_CHUNKS // 2],
                             src_pad[N_CHUNKS // 2:] + ACC_ROWS])
    dst_g = jnp.concatenate([dst_pad[:N_CHUNKS // 2],
                             dst_pad[N_CHUNKS // 2:] + ACC_ROWS])
    ea_pad = jnp.pad(edge_attr, ((0, npad), (0, 0)))

    p1, p2, p3, p4 = (params['enc1'], params['enc2'],
                      params['dec1'], params['dec2'])
    mlp = params['mlp']
    wecat = jnp.concatenate([p1['We'], p2['We'], p3['We'], p4['We']], axis=1)
    he1, he2, he3, he4 = _he_proj(ea_pad, wecat)

    gather = _gather_sc()
    merged = _merged_sc()

    t1 = jnp.concatenate(
        [x, jnp.ones((N_NODES, 1), jnp.float32)], axis=1)
    ga, gb = gather(t1, src_pad, dst_pad)
    vals = _score_tc(ga, gb, he1, _wpad(p1['Wl'], 8),
                     _wpad(p1['Wr']), _vpad(p1['att']), None, False)

    _, ga, gb = merged(vals, dst_pad, src_g, dst_g)
    vals = _score_tc(ga, gb, he2, _wpad(p2['Wl'], 2),
                     _wpad(p2['Wr']), _vpad(p2['att']), 8, True)

    tab2, ga, gb = merged(vals, dst_pad, src_g, dst_g)
    vals, edge_recon = _score_tc(
        ga, gb, he3, _wpad(p3['Wl'], 8), _wpad(p3['Wr']),
        _vpad(p3['att']), 2, False,
        mlp_w=(_wpad(mlp['W1'][0:2]), _wpad(mlp['W1'][2:4]),
               _vpad(mlp['b1']), mlp['W2'], mlp['b2'][None, :]))

    _, ga, gb = merged(vals, dst_pad, src_g, dst_g)
    vals = _score_tc(ga, gb, he4, _wpad(p4['Wl'], 15),
                     _wpad(p4['Wr']), _vpad(p4['att']), 8, True)

    acc4 = _scatter_sc()(vals, dst_pad)
    x_recon = _combine_final(acc4, 15)

    adj = _adj_head(tab2[0:ACC_ROWS])
    return x_recon, edge_recon, adj


# speedup vs baseline: 1.3509x; 1.3509x over previous
import functools

import jax
import jax.numpy as jnp
from jax import lax
from jax.experimental import pallas as pl
from jax.experimental.pallas import tpu as pltpu
from jax.experimental.pallas import tpu_sc as plsc

N_NODES = 10000
N_EDGES = 160000
CHUNK = 128
N_WORKERS = 32
CH_PER_W = 40
E_PAD = N_WORKERS * CH_PER_W * CHUNK
ACC_ROWS = 10112
ROWS_PER_TILE = ACC_ROWS // 16
W_ACC = 16

N_ADJ_BM = 256
N_MLP_BM = 4000

_SC_PARAMS = pltpu.CompilerParams(use_tc_tiling_on_sc=False)


@functools.lru_cache(maxsize=None)
def _gat_edge_sc(unused_f):
    mesh = plsc.VectorSubcoreMesh(core_axis_name="c", subcore_axis_name="s")

    @functools.partial(
        pl.kernel,
        out_type=jax.ShapeDtypeStruct((2, ACC_ROWS, W_ACC), jnp.float32),
        mesh=mesh,
        compiler_params=_SC_PARAMS,
        scratch_types=[
            pltpu.VMEM((CH_PER_W, CHUNK), jnp.int32),
            pltpu.VMEM((CH_PER_W, CHUNK), jnp.int32),
            pltpu.VMEM((2, CHUNK, 16), jnp.float32),
            pltpu.VMEM((2, CHUNK, 16), jnp.float32),
            pltpu.VMEM((2, CHUNK, 16), jnp.float32),
            pltpu.VMEM((2, CHUNK, 16), jnp.float32),
            pltpu.VMEM((ROWS_PER_TILE, 16), jnp.float32),
            pltpu.VMEM((16,), jnp.float32),
            pltpu.VMEM_SHARED((ACC_ROWS, W_ACC), jnp.float32),
            pltpu.SemaphoreType.DMA((2,)),
            pltpu.SemaphoreType.DMA((2,)),
        ],
    )
    def k(hl, hr, he, src2d, dst2d, att, out,
          src_v, dst_v, a_v, b_v, c_v, vals_v, bounce_v, att_v, accum,
          sem_ld, sem_sc):
        c = lax.axis_index("c")
        s = lax.axis_index("s")
        wid = c * 16 + s
        zeros16 = jnp.zeros((16,), jnp.float32)

        @pl.loop(0, ROWS_PER_TILE)
        def _zb(i):
            bounce_v[i, :] = zeros16

        row0 = s * ROWS_PER_TILE
        pltpu.sync_copy(bounce_v, accum.at[pl.ds(row0, ROWS_PER_TILE)])
        plsc.subcore_barrier()

        pltpu.sync_copy(att, att_v)
        attv = att_v[...]
        ch0 = wid * CH_PER_W
        pltpu.sync_copy(src2d.at[pl.ds(ch0, CH_PER_W)], src_v)
        pltpu.sync_copy(dst2d.at[pl.ds(ch0, CH_PER_W)], dst_v)
        base0 = ch0 * CHUNK

        def fire_loads(j):
            b = j & 1
            pltpu.async_copy(hl.at[src_v.at[j]], a_v.at[b], sem_ld.at[b])
            pltpu.async_copy(hr.at[dst_v.at[j]], b_v.at[b], sem_ld.at[b])
            pltpu.async_copy(he.at[pl.ds(base0 + j * CHUNK, CHUNK)],
                             c_v.at[b], sem_ld.at[b])

        def wait_loads(j):
            b = j & 1
            pltpu.make_async_copy(hl.at[src_v.at[j]], a_v.at[b],
                                  sem_ld.at[b]).wait()
            pltpu.make_async_copy(hr.at[dst_v.at[j]], b_v.at[b],
                                  sem_ld.at[b]).wait()
            pltpu.make_async_copy(he.at[pl.ds(base0 + j * CHUNK, CHUNK)],
                                  c_v.at[b], sem_ld.at[b]).wait()

        def wait_scatter(j):
            b = j & 1
            pltpu.make_async_copy(vals_v.at[b], accum.at[dst_v.at[j]],
                                  sem_sc.at[b]).wait()

        fire_loads(0)

        @pl.loop(0, CH_PER_W)
        def _chunk(j):
            b = j & 1

            @pl.when(j + 1 < CH_PER_W)
            def _():
                fire_loads(j + 1)

            wait_loads(j)

            @pl.when(j >= 2)
            def _():
                wait_scatter(j - 2)

            @pl.loop(0, CHUNK, unroll=16)
            def _e(e):
                ar = a_v[b, e, :]
                sg = ar + b_v[b, e, :] + c_v[b, e, :]
                lr = jnp.maximum(sg, jnp.float32(0.2) * sg)
                u = attv * lr
                w = u + lax.rev(u, (0,))
                sc = ((w[0] + w[1]) + (w[2] + w[3])
                      + (w[4] + w[5]) + (w[6] + w[7]))
                ex = jnp.exp(sc + zeros16)
                vals_v[b, e, :] = ex * ar

            pltpu.async_copy(vals_v.at[b], accum.at[dst_v.at[j]],
                             sem_sc.at[b], add=True)

        wait_scatter(CH_PER_W - 2)
        wait_scatter(CH_PER_W - 1)
        plsc.subcore_barrier()
        pltpu.sync_copy(accum.at[pl.ds(row0, ROWS_PER_TILE)], bounce_v)
        pltpu.sync_copy(bounce_v, out.at[c, pl.ds(row0, ROWS_PER_TILE)])

    return k


@functools.lru_cache(maxsize=None)
def _mlp_edge_sc():
    mesh = plsc.VectorSubcoreMesh(core_axis_name="c", subcore_axis_name="s")

    @functools.partial(
        pl.kernel,
        out_type=jax.ShapeDtypeStruct((E_PAD, 16), jnp.float32),
        mesh=mesh,
        compiler_params=_SC_PARAMS,
        scratch_types=[
            pltpu.VMEM((CH_PER_W, CHUNK), jnp.int32),
            pltpu.VMEM((CH_PER_W, CHUNK), jnp.int32),
            pltpu.VMEM((2, CHUNK, 16), jnp.float32),
            pltpu.VMEM((2, CHUNK, 16), jnp.float32),
            pltpu.VMEM((2, CHUNK, 16), jnp.float32),
            pltpu.VMEM((16,), jnp.float32),
            pltpu.SemaphoreType.DMA((2,)),
            pltpu.SemaphoreType.DMA((2,)),
        ],
    )
    def k(zp, zq, src2d, dst2d, b1, out,
          src_v, dst_v, a_v, b_v, vals_v, b1_v, sem_ld, sem_st):
        c = lax.axis_index("c")
        s = lax.axis_index("s")
        wid = c * 16 + s
        zeros16 = jnp.zeros((16,), jnp.float32)

        pltpu.sync_copy(b1, b1_v)
        b1v = b1_v[...]
        ch0 = wid * CH_PER_W
        pltpu.sync_copy(src2d.at[pl.ds(ch0, CH_PER_W)], src_v)
        pltpu.sync_copy(dst2d.at[pl.ds(ch0, CH_PER_W)], dst_v)
        base0 = ch0 * CHUNK

        def fire_loads(j):
            b = j & 1
            pltpu.async_copy(zp.at[src_v.at[j]], a_v.at[b], sem_ld.at[b])
            pltpu.async_copy(zq.at[dst_v.at[j]], b_v.at[b], sem_ld.at[b])

        def wait_loads(j):
            b = j & 1
            pltpu.make_async_copy(zp.at[src_v.at[j]], a_v.at[b],
                                  sem_ld.at[b]).wait()
            pltpu.make_async_copy(zq.at[dst_v.at[j]], b_v.at[b],
                                  sem_ld.at[b]).wait()

        def wait_store(j):
            b = j & 1
            pltpu.make_async_copy(
                vals_v.at[b], out.at[pl.ds(base0 + j * CHUNK, CHUNK)],
                sem_st.at[b]).wait()

        fire_loads(0)

        @pl.loop(0, CH_PER_W)
        def _chunk(j):
            b = j & 1

            @pl.when(j + 1 < CH_PER_W)
            def _():
                fire_loads(j + 1)

            wait_loads(j)

            @pl.when(j >= 2)
            def _():
                wait_store(j - 2)

            @pl.loop(0, CHUNK, unroll=16)
            def _e(e):
                vals_v[b, e, :] = jnp.maximum(
                    a_v[b, e, :] + b_v[b, e, :] + b1v, zeros16)

            pltpu.async_copy(vals_v.at[b],
                             out.at[pl.ds(base0 + j * CHUNK, CHUNK)],
                             sem_st.at[b])

        wait_store(CH_PER_W - 2)
        wait_store(CH_PER_W - 1)

    return k


def _padded(h, marker):
    n, fo = h.shape
    cols = [h, jnp.full((n, 1), marker, jnp.float32)]
    if fo < 15:
        cols.append(jnp.zeros((n, 15 - fo), jnp.float32))
    return jnp.concatenate(cols, axis=1)


def _nodes_body(x_ref, wl_ref, wr_ref, hl_ref, hr_ref):
    x = x_ref[...]
    hl_ref[...] = _padded(
        jnp.dot(x, wl_ref[...], preferred_element_type=jnp.float32), 1.0)
    hr_ref[...] = _padded(
        jnp.dot(x, wr_ref[...], preferred_element_type=jnp.float32), 0.0)


def _node_proj(x, wl, wr):
    n, fin = x.shape
    fo = wl.shape[1]
    return pl.pallas_call(
        _nodes_body,
        in_specs=[pl.BlockSpec((n, fin), lambda: (0, 0)),
                  pl.BlockSpec((fin, fo), lambda: (0, 0)),
                  pl.BlockSpec((fin, fo), lambda: (0, 0))],
        out_specs=[pl.BlockSpec((n, 16), lambda: (0, 0)),
                   pl.BlockSpec((n, 16), lambda: (0, 0))],
        out_shape=[jax.ShapeDtypeStruct((n, 16), jnp.float32),
                   jax.ShapeDtypeStruct((n, 16), jnp.float32)],
    )(x, wl, wr)


def _he_body(ea_ref, w_ref, o1_ref, o2_ref, o3_ref, o4_ref):
    h = jnp.dot(ea_ref[...], w_ref[...], preferred_element_type=jnp.float32)
    z1 = jnp.zeros((h.shape[0], 8), jnp.float32)
    o1_ref[...] = jnp.concatenate([h[:, 0:8], z1], axis=1)
    o2_ref[...] = jnp.concatenate([h[:, 8:10], z1, z1[:, 0:6]], axis=1)
    o3_ref[...] = jnp.concatenate([h[:, 10:18], z1], axis=1)
    o4_ref[...] = jnp.concatenate([h[:, 18:33], z1[:, 0:1]], axis=1)


def _he_proj(ea_pad, wecat):
    bm = 4096
    grid = (E_PAD // bm,)
    return pl.pallas_call(
        _he_body,
        grid=grid,
        in_specs=[pl.BlockSpec((bm, 35), lambda i: (i, 0)),
                  pl.BlockSpec((35, 33), lambda i: (0, 0))],
        out_specs=[pl.BlockSpec((bm, 16), lambda i: (i, 0))] * 4,
        out_shape=[jax.ShapeDtypeStruct((E_PAD, 16), jnp.float32)
                   for _ in range(4)],
    )(ea_pad, wecat)


def _combine(acc, f, act, weights, markers):
    n = N_NODES

    def body(acc_ref, *rest):
        w_refs = rest[:len(weights)]
        o_refs = rest[len(weights):]
        num = acc_ref[0, :n, 0:f] + acc_ref[1, :n, 0:f]
        den = acc_ref[0, :n, f:f + 1] + acc_ref[1, :n, f:f + 1]
        h = num / (den + jnp.float32(1e-16))
        if act:
            h = jnp.maximum(h, jnp.float32(0.0))
        o_refs[0][...] = h
        for w_ref, o_ref, m in zip(w_refs, o_refs[1:], markers):
            o_ref[...] = _padded(
                jnp.dot(h, w_ref[...], preferred_element_type=jnp.float32), m)

    out_shapes = [jax.ShapeDtypeStruct((n, f), jnp.float32)]
    in_specs = [pl.BlockSpec((2, ACC_ROWS, W_ACC), lambda: (0, 0, 0))]
    for w in weights:
        in_specs.append(pl.BlockSpec(w.shape, lambda: (0, 0)))
        out_shapes.append(jax.ShapeDtypeStruct((n, 16), jnp.float32))
    out_specs = [pl.BlockSpec(o.shape, lambda: (0, 0)) for o in out_shapes]
    return pl.pallas_call(
        body,
        in_specs=in_specs,
        out_specs=out_specs,
        out_shape=out_shapes,
    )(acc, *weights)


def _mlp_out_body(hmid_ref, w2_ref, b2_ref, out_ref):
    h = hmid_ref[...][:, 0:15]
    out_ref[...] = (jnp.dot(h, w2_ref[...], preferred_element_type=jnp.float32)
                    + b2_ref[...])


def _mlp_out(hmid, w2, b2):
    grid = (N_EDGES // N_MLP_BM,)
    return pl.pallas_call(
        _mlp_out_body,
        grid=grid,
        in_specs=[pl.BlockSpec((N_MLP_BM, 16), lambda i: (i, 0)),
                  pl.BlockSpec((15, 35), lambda i: (0, 0)),
                  pl.BlockSpec((1, 35), lambda i: (0, 0))],
        out_specs=pl.BlockSpec((N_MLP_BM, 35), lambda i: (i, 0)),
        out_shape=jax.ShapeDtypeStruct((N_EDGES, 35), jnp.float32),
    )(hmid, w2, b2)


def _adj_mean_body(z_ref, zt_ref, mean_ref):
    g = jnp.dot(z_ref[...], zt_ref[...], preferred_element_type=jnp.float32)
    mean_ref[...] = (jnp.sum(g, axis=1) / jnp.float32(g.shape[1]))[None, :]


def _adj_body(z_ref, zt_ref, mean_ref, out_ref):
    g = jnp.dot(z_ref[...], zt_ref[...], preferred_element_type=jnp.float32)
    out_ref[...] = jnp.tanh(g - mean_ref[...])


def _adj_head(z):
    n = z.shape[0]
    zt = z.T
    grid = (pl.cdiv(n, N_ADJ_BM),)
    means = pl.pallas_call(
        _adj_mean_body,
        grid=grid,
        in_specs=[
            pl.BlockSpec((N_ADJ_BM, 2), lambda i: (i, 0)),
            pl.BlockSpec((2, n), lambda i: (0, 0)),
        ],
        out_specs=pl.BlockSpec((1, N_ADJ_BM), lambda i: (0, i)),
        out_shape=jax.ShapeDtypeStruct((1, n), jnp.float32),
    )(z, zt)
    return pl.pallas_call(
        _adj_body,
        grid=grid,
        in_specs=[
            pl.BlockSpec((N_ADJ_BM, 2), lambda i: (i, 0)),
            pl.BlockSpec((2, n), lambda i: (0, 0)),
            pl.BlockSpec((1, n), lambda i: (0, 0)),
        ],
        out_specs=pl.BlockSpec((N_ADJ_BM, n), lambda i: (i, 0)),
        out_shape=jax.ShapeDtypeStruct((n, n), jnp.float32),
    )(z, zt, means)


def _pad16(v):
    return jnp.pad(v, (0, 16 - v.shape[0]))


def kernel(x, edge_index, edge_attr, params):
    src = edge_index[0]
    dst = edge_index[1]
    npad = E_PAD - N_EDGES
    src_pad = jnp.concatenate(
        [src, jnp.zeros((npad,), jnp.int32)]).reshape(-1, CHUNK)
    dst_pad = jnp.concatenate(
        [dst, jnp.full((npad,), N_NODES, jnp.int32)]).reshape(-1, CHUNK)
    ea_pad = jnp.pad(edge_attr, ((0, npad), (0, 0)))

    p1, p2, p3, p4 = (params['enc1'], params['enc2'],
                      params['dec1'], params['dec2'])
    mlp = params['mlp']
    wecat = jnp.concatenate([p1['We'], p2['We'], p3['We'], p4['We']], axis=1)
    he1, he2, he3, he4 = _he_proj(ea_pad, wecat)

    hl1, hr1 = _node_proj(x, p1['Wl'], p1['Wr'])
    acc1 = _gat_edge_sc(8)(hl1, hr1, he1, src_pad, dst_pad, _pad16(p1['att']))
    hl2, hr2 = _combine(acc1, 8, True, (p2['Wl'], p2['Wr']), (1.0, 0.0))[1:]
    acc2 = _gat_edge_sc(2)(hl2, hr2, he2, src_pad, dst_pad, _pad16(p2['att']))
    z, hl3, hr3, zp, zq = _combine(
        acc2, 2, False,
        (p3['Wl'], p3['Wr'], mlp['W1'][0:2], mlp['W1'][2:4]),
        (1.0, 0.0, 0.0, 0.0))
    acc3 = _gat_edge_sc(8)(hl3, hr3, he3, src_pad, dst_pad, _pad16(p3['att']))
    hl4, hr4 = _combine(acc3, 8, True, (p4['Wl'], p4['Wr']), (1.0, 0.0))[1:]
    acc4 = _gat_edge_sc(15)(hl4, hr4, he4, src_pad, dst_pad, _pad16(p4['att']))
    x_recon = _combine(acc4, 15, False, (), ())[0]

    hmid = _mlp_edge_sc()(zp, zq, src_pad, dst_pad, _pad16(mlp['b1']))
    edge_recon = _mlp_out(hmid, mlp['W2'], mlp['b2'][None, :])
    adj = _adj_head(z)
    return x_recon, edge_recon, adj
